# baseline (speedup 1.0000x reference)
import jax
import jax.numpy as jnp
from jax.experimental import pallas as pl


def _conv2d(x, w, b, stride, pad):
    out = jax.lax.conv_general_dilated(x, w, (stride, stride), [(pad, pad), (pad, pad)], dimension_numbers=('NCHW', 'OIHW', 'NCHW'))
    return out + b[None, :, None, None]


def _bn(x, g, b, m, v):
    shape = [1] * x.ndim
    shape[1] = -1
    return (x - m.reshape(shape)) / jnp.sqrt(v.reshape(shape) + 1e-5) * g.reshape(shape) + b.reshape(shape)


def _adaptive_avg_pool(x, out_hw):
    H, W = x.shape[2], x.shape[3]
    rows = []
    for i in range(out_hw):
        hs = (i * H) // out_hw
        he = -((-(i + 1) * H) // out_hw)
        cols = []
        for j in range(out_hw):
            ws = (j * W) // out_hw
            we = -((-(j + 1) * W) // out_hw)
            cols.append(jnp.mean(x[:, :, hs:he, ws:we], axis=(2, 3)))
        rows.append(jnp.stack(cols, axis=-1))
    return jnp.stack(rows, axis=-2)


def _identity_kernel(a_ref, o_ref):
    o_ref[...] = a_ref[...]


def _pl_identity(a):
    return pl.pallas_call(
        _identity_kernel,
        out_shape=jax.ShapeDtypeStruct(a.shape, a.dtype),
    )(a)


def kernel(x, conv1_w, conv1_b, bn1c_g, bn1c_b, bn1c_m, bn1c_v, conv2_w, conv2_b, bn2c_g, bn2c_b, bn2c_m, bn2c_v, fc1_w, fc1_b, bn1d_g, bn1d_b, bn1d_m, bn1d_v, fc2_w, fc2_b, temperature):
    h = x.astype(jnp.float32)
    h = _conv2d(h, conv1_w, conv1_b, 2, 2)
    h = _bn(h, bn1c_g, bn1c_b, bn1c_m, bn1c_v)
    h = jax.nn.relu(h)
    h = jax.lax.reduce_window(h, -jnp.inf, jax.lax.max, (1, 1, 3, 3), (1, 1, 2, 2), 'VALID')
    h = _conv2d(h, conv2_w, conv2_b, 2, 1)
    h = _bn(h, bn2c_g, bn2c_b, bn2c_m, bn2c_v)
    h = jax.nn.relu(h)
    h = _adaptive_avg_pool(h, 3)
    h = h.reshape(h.shape[0], -1)
    h = h @ fc1_w.T + fc1_b
    h = _bn(h, bn1d_g, bn1d_b, bn1d_m, bn1d_v)
    h = jax.nn.relu(h)
    logits = h @ fc2_w.T + fc2_b
    logits = logits / jnp.clip(temperature, 0.1, 3.0)
    vals, idx = jax.lax.top_k(logits, 2)
    g = jax.nn.softmax(vals, axis=-1)
    g = g / (jnp.sum(g, axis=-1, keepdims=True) + 1e-10)
    gates = jnp.zeros_like(logits).at[jnp.arange(logits.shape[0])[:, None], idx].set(g)
    gates = _pl_identity(gates)
    return gates, idx, logits


# speedup vs baseline: 1.1996x; 1.1996x over previous
import jax
import jax.numpy as jnp
from jax.experimental import pallas as pl
from jax.experimental.pallas import tpu as pltpu

_B = 16
_E = 16


def _sel1(dx):
    w = jax.lax.broadcasted_iota(jnp.int32, (256, 112), 0)
    j = jax.lax.broadcasted_iota(jnp.int32, (256, 112), 1)
    return (w == 2 * j + dx).astype(jnp.float32)


def _sel2(dx):
    w = jax.lax.broadcasted_iota(jnp.int32, (128, 28), 0)
    j = jax.lax.broadcasted_iota(jnp.int32, (128, 28), 1)
    return ((w == 4 * j + 2 * dx - 2) & (w <= 108)).astype(jnp.float32)


def _pool_cols():
    j = jax.lax.broadcasted_iota(jnp.int32, (28, 3), 0)
    b = jax.lax.broadcasted_iota(jnp.int32, (28, 3), 1)
    lo, hi = (b * 28) // 3, ((b + 1) * 28 + 2) // 3
    return ((j >= lo) & (j < hi)).astype(jnp.float32) * 0.1


def _pool_rows():
    r = jax.lax.broadcasted_iota(jnp.int32, (1792, 192), 0)
    c = jax.lax.broadcasted_iota(jnp.int32, (1792, 192), 1)
    o2, i2 = r // 28, r % 28
    bi, o2c = c // 64, c % 64
    lo, hi = (bi * 28) // 3, ((bi + 1) * 28 + 2) // 3
    return ((o2 == o2c) & (i2 >= lo) & (i2 < hi)).astype(jnp.float32) * 0.1


def _gate_kernel(x_ref, st_ref, w1_ref, b1_ref, w2_ref, b2_ref, fc1_ref,
                 fb1_ref, fc2_ref, fb2_ref, t_ref, gates_ref, idx_ref,
                 logits_ref, xlo_ref, xhi_ref, hs_ref, cmp_ref, feats_ref):
    b = pl.program_id(0)
    h = pl.program_id(1)
    f32 = jnp.float32
    xb = x_ref[0]
    st = st_ref[0]

    @pl.when(h == 0)
    def _():
        xlo_ref[:, 0:2, :] = jnp.zeros((96, 2, 128), f32)
        xlo_ref[:, 114:115, 2:128] = st[:, 0:1, 0:126]
        xhi_ref[:, 0:2, :] = jnp.zeros((96, 2, 128), f32)
        xhi_ref[:, 114:115, 0:98] = st[:, 0:1, 126:224]

    @pl.when(h == 1)
    def _():
        xlo_ref[:, 0:2, 2:128] = st[:, 6:8, 0:126]
        xlo_ref[:, 114:115, :] = jnp.zeros((96, 1, 128), f32)
        xhi_ref[:, 0:2, 0:98] = st[:, 6:8, 126:224]
        xhi_ref[:, 114:115, :] = jnp.zeros((96, 1, 128), f32)

    xlo_ref[:, :, 0:2] = jnp.zeros((96, 115, 2), f32)
    xlo_ref[:, 2:114, 2:128] = xb[:, :, 0:126]
    xhi_ref[:, :, 98:128] = jnp.zeros((96, 115, 30), f32)
    xhi_ref[:, 2:114, 0:98] = xb[:, :, 126:224]

    u = jnp.zeros((160, 56 * 256), f32)
    for dy in range(5):
        lo = xlo_ref[:, pl.Slice(dy, 56, 2), :]
        hi = xhi_ref[:, pl.Slice(dy, 56, 2), :]
        r = jnp.concatenate([lo, hi], axis=2)
        u = u + jnp.dot(w1_ref[dy], r.reshape(96, 56 * 256),
                        preferred_element_type=f32)
    acc = jnp.zeros((1792, 112), f32)
    for dx in range(5):
        ud = u[32 * dx:32 * dx + 32, :].reshape(32, 56, 256)
        acc = acc + jnp.dot(ud.reshape(1792, 256), _sel1(dx),
                            preferred_element_type=f32)
    hv = jnp.maximum(acc + b1_ref[...], 0.0)
    hs_ref[:, pl.ds(56 * h, 56), 0:112] = hv.reshape(32, 56, 112)
    hs_ref[:, pl.ds(56 * h, 56), 112:128] = jnp.zeros((32, 56, 16), f32)

    @pl.when(h == 1)
    def _():
        rp = jnp.maximum(
            jnp.maximum(hs_ref[:, pl.Slice(0, 55, 2), :],
                        hs_ref[:, pl.Slice(1, 55, 2), :]),
            hs_ref[:, pl.Slice(2, 55, 2), :])
        cm = jnp.maximum(jnp.maximum(rp[:, :, 0:126], rp[:, :, 1:127]),
                         rp[:, :, 2:128])
        cmp_ref[:, :, 126:128] = jnp.zeros((32, 57, 2), f32)
        cmp_ref[:, 0:1, :] = jnp.zeros((32, 1, 128), f32)
        cmp_ref[:, 56:57, :] = jnp.zeros((32, 1, 128), f32)
        cmp_ref[:, 1:56, 0:126] = cm

        r2 = jnp.concatenate([cmp_ref[:, pl.Slice(dy, 28, 2), :]
                              for dy in range(3)], axis=0)
        u2 = jnp.dot(w2_ref[...], r2.reshape(96, 28 * 128),
                     preferred_element_type=f32)
        acc2 = jnp.zeros((1792, 28), f32)
        for dx in range(3):
            ud = u2[64 * dx:64 * dx + 64, :].reshape(64, 28, 128)
            acc2 = acc2 + jnp.dot(ud.reshape(1792, 128), _sel2(dx),
                                  preferred_element_type=f32)
        h2 = jnp.maximum(acc2 + b2_ref[...], 0.0)

        cb = jax.lax.dot_general(_pool_cols(), h2, (((0,), (1,)), ((), ())),
                                 preferred_element_type=f32)
        rb = jnp.dot(cb, _pool_rows(), preferred_element_type=f32)
        feats = jnp.concatenate([rb[:, 64 * bi:64 * bi + 64]
                                 for bi in range(3)], axis=0)
        feats_ref[:, pl.ds(b, 1), :] = feats.reshape(9, 1, 64)

        @pl.when(b == _B - 1)
        def _():
            h3 = fb1_ref[...]
            for t in range(9):
                h3 = h3 + jnp.dot(feats_ref[t], fc1_ref[t],
                                  preferred_element_type=f32)
            h3 = jnp.maximum(h3, 0.0)
            lg = jnp.dot(h3, fc2_ref[...],
                         preferred_element_type=f32) + fb2_ref[...]
            lg = lg / jnp.clip(t_ref[0, 0], 0.1, 3.0)
            col = jax.lax.broadcasted_iota(jnp.int32, (_B, _E), 1)
            m0 = jnp.max(lg, axis=1, keepdims=True)
            i0 = jnp.min(jnp.where(lg == m0, col, _E), axis=1, keepdims=True)
            msk = jnp.where(col == i0, -jnp.inf, lg)
            m1 = jnp.max(msk, axis=1, keepdims=True)
            i1 = jnp.min(jnp.where(msk == m1, col, _E), axis=1,
                         keepdims=True)
            e1 = jnp.exp(m1 - m0)
            g0 = 1.0 / (1.0 + e1)
            g1 = e1 / (1.0 + e1)
            rs = g0 + g1 + 1e-10
            g0, g1 = g0 / rs, g1 / rs
            gates_ref[...] = (jnp.where(col == i0, g0, 0.0)
                              + jnp.where(col == i1, g1, 0.0))
            idx_ref[...] = jnp.concatenate([i0, i1], axis=1)
            logits_ref[...] = lg


def kernel(x, conv1_w, conv1_b, bn1c_g, bn1c_b, bn1c_m, bn1c_v, conv2_w,
           conv2_b, bn2c_g, bn2c_b, bn2c_m, bn2c_v, fc1_w, fc1_b, bn1d_g,
           bn1d_b, bn1d_m, bn1d_v, fc2_w, fc2_b, temperature):
    f32 = jnp.float32
    x = x.astype(f32)
    s1 = bn1c_g / jnp.sqrt(bn1c_v + 1e-5)
    w1t = (conv1_w * s1[:, None, None, None]).transpose(3, 0, 2, 1)
    w1t = w1t.reshape(160, 5, 96).transpose(1, 0, 2)
    b1e = jnp.repeat(((conv1_b - bn1c_m) * s1 + bn1c_b).reshape(32, 1),
                     56, axis=0)
    s2 = bn2c_g / jnp.sqrt(bn2c_v + 1e-5)
    w2t = (conv2_w * s2[:, None, None, None]).transpose(3, 0, 2, 1)
    w2t = w2t.reshape(192, 96)
    b2e = jnp.repeat(((conv2_b - bn2c_m) * s2 + bn2c_b).reshape(64, 1),
                     28, axis=0)
    s3 = bn1d_g / jnp.sqrt(bn1d_v + 1e-5)
    fc1t = (fc1_w * s3[:, None]).reshape(128, 64, 9).transpose(2, 1, 0)
    fb1 = ((fc1_b - bn1d_m) * s3 + bn1d_b).reshape(1, 128)
    fc2t = fc2_w.T
    fb2 = fc2_b.reshape(1, _E)
    temp = jnp.asarray(temperature, f32).reshape(1, 1)

    gates, idx, logits = pl.pallas_call(
        _gate_kernel,
        grid=(_B, 2),
        in_specs=[
            pl.BlockSpec((1, 96, 112, 224), lambda b, h: (b, 0, h, 0)),
            pl.BlockSpec((1, 96, 8, 224), lambda b, h: (b, 0, 14 - h, 0)),
            pl.BlockSpec((5, 160, 96), lambda b, h: (0, 0, 0)),
            pl.BlockSpec((1792, 1), lambda b, h: (0, 0)),
            pl.BlockSpec((192, 96), lambda b, h: (0, 0)),
            pl.BlockSpec((1792, 1), lambda b, h: (0, 0)),
            pl.BlockSpec((9, 64, 128), lambda b, h: (0, 0, 0)),
            pl.BlockSpec((1, 128), lambda b, h: (0, 0)),
            pl.BlockSpec((128, 16), lambda b, h: (0, 0)),
            pl.BlockSpec((1, _E), lambda b, h: (0, 0)),
            pl.BlockSpec((1, 1), lambda b, h: (0, 0)),
        ],
        out_specs=[
            pl.BlockSpec((_B, _E), lambda b, h: (0, 0)),
            pl.BlockSpec((_B, 2), lambda b, h: (0, 0)),
            pl.BlockSpec((_B, _E), lambda b, h: (0, 0)),
        ],
        out_shape=[
            jax.ShapeDtypeStruct((_B, _E), f32),
            jax.ShapeDtypeStruct((_B, 2), jnp.int32),
            jax.ShapeDtypeStruct((_B, _E), f32),
        ],
        scratch_shapes=[
            pltpu.VMEM((96, 115, 128), f32),
            pltpu.VMEM((96, 115, 128), f32),
            pltpu.VMEM((32, 112, 128), f32),
            pltpu.VMEM((32, 57, 128), f32),
            pltpu.VMEM((9, _B, 64), f32),
        ],
    )(x, x, w1t, b1e, w2t, b2e, fc1t, fb1, fc2t, fb2, temp)
    return gates, idx, logits
